# Initial kernel scaffold; baseline (speedup 1.0000x reference)
import jax
import jax.numpy as jnp
from jax.experimental import pallas as pl


def kernel(feat, edge_weight, W_self, b_self, W_first, b_first, W_second, b_second, Wh1, bh1, Wh2, bh2, W_fusion, W_conv, protein_idx, hyperedge_idx, edge_type):
    raise NotImplementedError("write your pallas kernel here")


# speedup vs baseline: 12.1827x; 12.1827x over previous
import functools

import jax
import jax.numpy as jnp
from jax import lax
from jax.experimental import pallas as pl
from jax.experimental.pallas import tpu as pltpu
from jax.experimental.pallas import tpu_sc as plsc

N_P = 10000
N_H = 10000
N_I = 160000
D = 128
DH = 64
HD = 32
NHEADS = 4
NC = 2
NS = 16
EPT = N_I // NS
CH = 40
NCHT = EPT // CH
NBUF = 5
NE_G = NBUF * CH
NGR = NCHT // NBUF
NGR2 = NGR // 2
ZR = 128
NROWS = 2 * N_H
NROWSP = 20480
STRIPE = NROWSP // NS


def _seg_pass(tab, gidx_cat, dst4d, w):
    mesh = plsc.VectorSubcoreMesh(core_axis_name="c", subcore_axis_name="s")

    @functools.partial(
        pl.kernel,
        mesh=mesh,
        compiler_params=pltpu.CompilerParams(
            needs_layout_passes=False, use_tc_tiling_on_sc=False),
        out_type=jax.ShapeDtypeStruct((NC, NROWSP, DH), jnp.float32),
        scratch_types=[
            pltpu.VMEM((2, NE_G), jnp.int32),
            pltpu.VMEM((2, NBUF, CH), jnp.int32),
            pltpu.VMEM((2, NE_G), jnp.float32),
            pltpu.VMEM((NBUF, CH, DH), jnp.float32),
            pltpu.VMEM((ZR, DH), jnp.float32),
            pltpu.VMEM_SHARED((NROWSP, DH), jnp.float32),
            pltpu.SemaphoreType.DMA,
            pltpu.SemaphoreType.DMA,
            pltpu.SemaphoreType.DMA,
        ],
    )
    def seg(tab_hbm, gidx_hbm, dst_hbm, w_hbm, out_hbm,
            gidx_v, dst_v, w_v, rows_v, zero_v, acc_sh, isem, gsem, ssem):
        c = lax.axis_index("c")
        s = lax.axis_index("s")
        ebase = c * N_I + s * EPT

        def issue_idx(g, p):
            pltpu.async_copy(
                gidx_hbm.at[pl.ds(ebase + g * NE_G, NE_G)], gidx_v.at[p], isem)
            pltpu.async_copy(dst_hbm.at[s, g], dst_v.at[p], isem)
            pltpu.async_copy(
                w_hbm.at[pl.ds(s * EPT + g * NE_G, NE_G)], w_v.at[p], isem)

        def wait_idx(p):
            pltpu.make_async_copy(
                gidx_hbm.at[pl.ds(0, NE_G)], gidx_v.at[p], isem).wait()
            pltpu.make_async_copy(dst_hbm.at[s, 0], dst_v.at[p], isem).wait()
            pltpu.make_async_copy(
                w_hbm.at[pl.ds(0, NE_G)], w_v.at[p], isem).wait()

        issue_idx(0, 0)
        issue_idx(1, 1)

        zv = jnp.zeros((16,), jnp.float32)

        def zrow(i, carry):
            for t in range(DH // 16):
                zero_v[i, pl.ds(t * 16, 16)] = zv
            return carry

        lax.fori_loop(0, ZR, zrow, 0)
        rbase = s * STRIPE
        for z in range(STRIPE // ZR):
            pltpu.sync_copy(zero_v, acc_sh.at[pl.ds(rbase + z * ZR, ZR)])
        plsc.subcore_barrier()

        def scale_chunk(p, b):
            wbase = b * CH

            def ebody(j, carry):
                wj = plsc.load_gather(
                    w_v.at[p], [jnp.full((16,), wbase + j, jnp.int32)])
                for t in range(DH // 16):
                    sl = rows_v[b, j, pl.ds(t * 16, 16)]
                    rows_v[b, j, pl.ds(t * 16, 16)] = sl * wj
                return carry

            lax.fori_loop(0, CH, ebody, 0)

        def run_group(g, p):
            wait_idx(p)
            gds = []
            for b in range(NBUF):
                gds.append(pltpu.async_copy(
                    tab_hbm.at[gidx_v.at[p, pl.ds(b * CH, CH)]],
                    rows_v.at[b], gsem))
            sds = []
            for b in range(NBUF):
                gds[b].wait()
                scale_chunk(p, b)
                sds.append(pltpu.async_copy(
                    rows_v.at[b], acc_sh.at[dst_v.at[p, b]], ssem, add=True))
            for d_ in sds:
                d_.wait()

            @pl.when(g + 2 < NGR)
            def _():
                issue_idx(g + 2, p)

        def group_pair(gp, carry):
            run_group(gp * 2, 0)
            run_group(gp * 2 + 1, 1)
            return carry

        lax.fori_loop(0, NGR2, group_pair, 0)
        plsc.subcore_barrier()
        pltpu.sync_copy(acc_sh.at[pl.ds(rbase, STRIPE)],
                        out_hbm.at[c, pl.ds(rbase, STRIPE)])

    return seg(tab, gidx_cat, dst4d, w)


def _attn_body(he_ref, wh1_ref, bh1_ref, wh2_ref, bh2_ref, wf_ref, out_ref):
    lo = he_ref[0]
    hi = he_ref[1]
    acc = None
    for k in range(NHEADS):
        t = jnp.dot(lo, wh1_ref[k, :DH, :], preferred_element_type=jnp.float32)
        t = t + jnp.dot(hi, wh1_ref[k, DH:, :], preferred_element_type=jnp.float32)
        t = jnp.maximum(t + bh1_ref[k][None, :], 0.0)
        sk = jax.nn.sigmoid(
            jnp.sum(t * wh2_ref[k][None, :], axis=1, keepdims=True)
            + bh2_ref[k, 0])
        contrib = sk * wf_ref[k, 0]
        acc = contrib if acc is None else acc + contrib
    out_ref[0] = lo * acc
    out_ref[1] = hi * acc


def _attn(he2, Wh1, bh1, Wh2, bh2, Wf):
    RB = 2048
    grid = NROWSP // RB
    return pl.pallas_call(
        _attn_body,
        grid=(grid,),
        in_specs=[
            pl.BlockSpec((NC, RB, DH), lambda i: (0, i, 0)),
            pl.BlockSpec((NHEADS, D, HD), lambda i: (0, 0, 0)),
            pl.BlockSpec((NHEADS, HD), lambda i: (0, 0)),
            pl.BlockSpec((NHEADS, HD), lambda i: (0, 0)),
            pl.BlockSpec((NHEADS, 1), lambda i: (0, 0)),
            pl.BlockSpec((NHEADS, 1), lambda i: (0, 0)),
        ],
        out_specs=pl.BlockSpec((NC, RB, DH), lambda i: (0, i, 0)),
        out_shape=jax.ShapeDtypeStruct((NC, NROWSP, DH), jnp.float32),
    )(he2, Wh1, bh1, Wh2, bh2, Wf)


def _final_body(feat_ref, a0_ref, a1_ref, ws_ref, bs_ref, wf_ref, bf_ref,
                wsec_ref, bsec_ref, wc_ref, out_ref):
    f = feat_ref[...]
    f0 = jnp.dot(f, ws_ref[...], preferred_element_type=jnp.float32) + bs_ref[0][None, :]
    f1 = (jnp.dot(a0_ref[0], wf_ref[:DH, :], preferred_element_type=jnp.float32)
          + jnp.dot(a0_ref[1], wf_ref[DH:, :], preferred_element_type=jnp.float32)
          + bf_ref[0][None, :])
    f2 = (jnp.dot(a1_ref[0], wsec_ref[:DH, :], preferred_element_type=jnp.float32)
          + jnp.dot(a1_ref[1], wsec_ref[DH:, :], preferred_element_type=jnp.float32)
          + bsec_ref[0][None, :])
    p0 = jnp.mean(f0, axis=1, keepdims=True)
    p1 = jnp.mean(f1, axis=1, keepdims=True)
    p2 = jnp.mean(f2, axis=1, keepdims=True)
    wc = wc_ref[...]
    a_list = []
    for i in range(3):
        a_list.append(jax.nn.sigmoid(p0 * wc[i, 0] + p1 * wc[i, 1] + p2 * wc[i, 2]))
    fused = f0 * a_list[0] + f1 * a_list[1] + f2 * a_list[2]
    out_ref[...] = jnp.maximum(fused + f, 0.0)


def _final(feat, agg, W_self, b_self, W_first, b_first, W_second, b_second, W_conv):
    RB = 2000
    grid = N_P // RB
    b_self2 = b_self.reshape(1, D)
    b_first2 = b_first.reshape(1, D)
    b_second2 = b_second.reshape(1, D)
    return pl.pallas_call(
        _final_body,
        grid=(grid,),
        in_specs=[
            pl.BlockSpec((RB, D), lambda i: (i, 0)),
            pl.BlockSpec((NC, RB, DH), lambda i: (0, i, 0)),
            pl.BlockSpec((NC, RB, DH), lambda i: (0, i + N_P // RB, 0)),
            pl.BlockSpec((D, D), lambda i: (0, 0)),
            pl.BlockSpec((1, D), lambda i: (0, 0)),
            pl.BlockSpec((D, D), lambda i: (0, 0)),
            pl.BlockSpec((1, D), lambda i: (0, 0)),
            pl.BlockSpec((D, D), lambda i: (0, 0)),
            pl.BlockSpec((1, D), lambda i: (0, 0)),
            pl.BlockSpec((3, 3), lambda i: (0, 0)),
        ],
        out_specs=pl.BlockSpec((RB, D), lambda i: (i, 0)),
        out_shape=jax.ShapeDtypeStruct((N_P, D), jnp.float32),
    )(feat, agg, agg, W_self, b_self2, W_first, b_first2,
      W_second, b_second2, W_conv)


def kernel(feat, edge_weight, W_self, b_self, W_first, b_first, W_second,
           b_second, Wh1, bh1, Wh2, bh2, W_fusion, W_conv,
           protein_idx, hyperedge_idx, edge_type):
    pid = protein_idx.astype(jnp.int32)
    hid = hyperedge_idx.astype(jnp.int32)
    et = edge_type.astype(jnp.int32)
    w = edge_weight[:, 0]

    dst1 = hid + et * N_H
    s2 = pid + et * N_P
    gA = jnp.concatenate([pid, pid + N_P])
    gC = jnp.concatenate([dst1, dst1 + NROWSP])
    dst1_2d = dst1.reshape(NS, NGR, NBUF, CH)
    s2_2d = s2.reshape(NS, NGR, NBUF, CH)
    featT = feat.reshape(N_P, NC, DH).transpose(1, 0, 2).reshape(NC * N_P, DH)

    he2 = _seg_pass(featT, gA, dst1_2d, w)
    hew2 = _attn(he2, Wh1, bh1, Wh2.reshape(NHEADS, HD), bh2, W_fusion)
    agg = _seg_pass(hew2.reshape(NC * NROWSP, DH), gC, s2_2d, w)
    return _final(feat, agg, W_self, b_self, W_first, b_first,
                  W_second, b_second, W_conv)


# speedup vs baseline: 19.0307x; 1.5621x over previous
import functools

import jax
import jax.numpy as jnp
from jax import lax
from jax.experimental import pallas as pl
from jax.experimental.pallas import tpu as pltpu
from jax.experimental.pallas import tpu_sc as plsc

N_P = 10000
N_H = 10000
N_I = 160000
D = 128
DH = 64
HD = 32
NHEADS = 4
NC = 2
NS = 16
EPT = N_I // NS
CH = 40
NCHT = EPT // CH
NBUF = 5
NBUF2 = 2 * NBUF
NE_G = NBUF * CH
NGR = NCHT // NBUF
NGR2 = NGR // 2
ZR = 64
NROWS = 2 * N_H
NROWSP = 20480
STRIPE = NROWSP // NS


def _seg_pass(tab, gidx_cat, dst4d, w):
    mesh = plsc.VectorSubcoreMesh(core_axis_name="c", subcore_axis_name="s")

    @functools.partial(
        pl.kernel,
        mesh=mesh,
        compiler_params=pltpu.CompilerParams(
            needs_layout_passes=False, use_tc_tiling_on_sc=False),
        out_type=jax.ShapeDtypeStruct((NC, NROWSP, DH), jnp.float32),
        scratch_types=[
            pltpu.VMEM((EPT,), jnp.int32),
            pltpu.VMEM((2, NBUF, CH), jnp.int32),
            pltpu.VMEM((2, NE_G + 16), jnp.float32),
            pltpu.VMEM((NBUF2, CH, DH), jnp.float32),
            pltpu.VMEM((ZR, DH), jnp.float32),
            pltpu.VMEM_SHARED((NROWSP, DH), jnp.float32),
            pltpu.SemaphoreType.DMA,
            pltpu.SemaphoreType.DMA,
            pltpu.SemaphoreType.DMA,
        ],
    )
    def seg(tab_hbm, gidx_hbm, dst_hbm, w_hbm, out_hbm,
            gidx_v, dst_v, w_v, rows_v, zero_v, acc_sh, dsem, gsem, ssem):
        c = lax.axis_index("c")
        s = lax.axis_index("s")
        ebase = c * N_I + s * EPT
        pltpu.sync_copy(gidx_hbm.at[pl.ds(ebase, EPT)], gidx_v)

        zv = jnp.zeros((16,), jnp.float32)

        def zrow(i, carry):
            for t in range(DH // 16):
                zero_v[i, pl.ds(t * 16, 16)] = zv
            return carry

        lax.fori_loop(0, ZR, zrow, 0)
        rbase = s * STRIPE
        for z in range(STRIPE // ZR):
            pltpu.sync_copy(zero_v, acc_sh.at[pl.ds(rbase + z * ZR, ZR)])
        plsc.subcore_barrier()

        def issue_group(g, h):
            pltpu.async_copy(dst_hbm.at[s, g], dst_v.at[h], dsem)
            pltpu.async_copy(
                w_hbm.at[pl.ds(s * EPT + g * NE_G, NE_G)],
                w_v.at[h, pl.ds(0, NE_G)], dsem)
            for b in range(NBUF):
                B = h * NBUF + b
                pltpu.async_copy(
                    tab_hbm.at[gidx_v.at[pl.ds(g * NE_G + b * CH, CH)]],
                    rows_v.at[B], gsem)

        def scale_buf(h, b):
            B = h * NBUF + b
            def ebody(jj, carry):
                j0 = jj * 8
                w16 = w_v[h, pl.ds(b * CH + j0, 16)]
                for u in range(8):
                    wj = jnp.take_along_axis(
                        w16, jnp.full((16,), u, jnp.int32), axis=0)
                    j = j0 + u
                    for t in range(DH // 16):
                        sl = rows_v[B, j, pl.ds(t * 16, 16)]
                        rows_v[B, j, pl.ds(t * 16, 16)] = sl * wj
                return carry

            lax.fori_loop(0, CH // 8, ebody, 0)

        def process_group(h):
            pltpu.make_async_copy(dst_hbm.at[s, 0], dst_v.at[h], dsem).wait()
            pltpu.make_async_copy(w_hbm.at[pl.ds(0, NE_G)],
                                  w_v.at[h, pl.ds(0, NE_G)], dsem).wait()
            for b in range(NBUF):
                B = h * NBUF + b
                pltpu.make_async_copy(
                    tab_hbm.at[gidx_v.at[pl.ds(0, CH)]], rows_v.at[B],
                    gsem).wait()
                scale_buf(h, b)
                pltpu.async_copy(
                    rows_v.at[B], acc_sh.at[dst_v.at[h, b]], ssem, add=True)

        def drain_group(h):
            for b in range(NBUF):
                B = h * NBUF + b
                pltpu.make_async_copy(
                    rows_v.at[B], acc_sh.at[dst_v.at[h, b]], ssem).wait()

        def step(g, h, drain):
            if drain:
                drain_group(h)
            issue_group(g, h)
            process_group(1 - h)

        issue_group(0, 0)

        step(1, 1, False)
        step(2, 0, True)

        def pair_body(q, carry):
            g1 = q * 2 + 1
            step(g1, 1, True)
            step(g1 + 1, 0, True)
            return carry

        lax.fori_loop(1, NGR2 - 1, pair_body, 0)
        step(NGR - 1, 1, True)
        process_group(1)
        drain_group(0)
        drain_group(1)

        plsc.subcore_barrier()
        pltpu.sync_copy(acc_sh.at[pl.ds(rbase, STRIPE)],
                        out_hbm.at[c, pl.ds(rbase, STRIPE)])

    return seg(tab, gidx_cat, dst4d, w)


def _attn_body(he_ref, wh1_ref, bh1_ref, wh2_ref, bh2_ref, wf_ref, out_ref):
    lo = he_ref[0]
    hi = he_ref[1]
    acc = None
    for k in range(NHEADS):
        t = jnp.dot(lo, wh1_ref[k, :DH, :], preferred_element_type=jnp.float32)
        t = t + jnp.dot(hi, wh1_ref[k, DH:, :], preferred_element_type=jnp.float32)
        t = jnp.maximum(t + bh1_ref[k][None, :], 0.0)
        sk = jax.nn.sigmoid(
            jnp.sum(t * wh2_ref[k][None, :], axis=1, keepdims=True)
            + bh2_ref[k, 0])
        contrib = sk * wf_ref[k, 0]
        acc = contrib if acc is None else acc + contrib
    out_ref[0] = lo * acc
    out_ref[1] = hi * acc


def _attn(he2, Wh1, bh1, Wh2, bh2, Wf):
    RB = 2048
    grid = NROWSP // RB
    return pl.pallas_call(
        _attn_body,
        grid=(grid,),
        in_specs=[
            pl.BlockSpec((NC, RB, DH), lambda i: (0, i, 0)),
            pl.BlockSpec((NHEADS, D, HD), lambda i: (0, 0, 0)),
            pl.BlockSpec((NHEADS, HD), lambda i: (0, 0)),
            pl.BlockSpec((NHEADS, HD), lambda i: (0, 0)),
            pl.BlockSpec((NHEADS, 1), lambda i: (0, 0)),
            pl.BlockSpec((NHEADS, 1), lambda i: (0, 0)),
        ],
        out_specs=pl.BlockSpec((NC, RB, DH), lambda i: (0, i, 0)),
        out_shape=jax.ShapeDtypeStruct((NC, NROWSP, DH), jnp.float32),
    )(he2, Wh1, bh1, Wh2, bh2, Wf)


def _feat0_body(feat_ref, ws_ref, bs_ref, out_ref):
    out_ref[...] = (jnp.dot(feat_ref[...], ws_ref[...],
                            preferred_element_type=jnp.float32)
                    + bs_ref[0][None, :])


def _feat0(feat, W_self, b_self):
    RB = 2000
    return pl.pallas_call(
        _feat0_body,
        grid=(N_P // RB,),
        in_specs=[
            pl.BlockSpec((RB, D), lambda i: (i, 0)),
            pl.BlockSpec((D, D), lambda i: (0, 0)),
            pl.BlockSpec((1, D), lambda i: (0, 0)),
        ],
        out_specs=pl.BlockSpec((RB, D), lambda i: (i, 0)),
        out_shape=jax.ShapeDtypeStruct((N_P, D), jnp.float32),
    )(feat, W_self, b_self.reshape(1, D))


def _final_body(feat_ref, f0_ref, a0_ref, a1_ref, wf_ref, bf_ref,
                wsec_ref, bsec_ref, wc_ref, out_ref):
    f = feat_ref[...]
    f0 = f0_ref[...]
    f1 = (jnp.dot(a0_ref[0], wf_ref[:DH, :], preferred_element_type=jnp.float32)
          + jnp.dot(a0_ref[1], wf_ref[DH:, :], preferred_element_type=jnp.float32)
          + bf_ref[0][None, :])
    f2 = (jnp.dot(a1_ref[0], wsec_ref[:DH, :], preferred_element_type=jnp.float32)
          + jnp.dot(a1_ref[1], wsec_ref[DH:, :], preferred_element_type=jnp.float32)
          + bsec_ref[0][None, :])
    p0 = jnp.mean(f0, axis=1, keepdims=True)
    p1 = jnp.mean(f1, axis=1, keepdims=True)
    p2 = jnp.mean(f2, axis=1, keepdims=True)
    wc = wc_ref[...]
    a_list = []
    for i in range(3):
        a_list.append(jax.nn.sigmoid(p0 * wc[i, 0] + p1 * wc[i, 1] + p2 * wc[i, 2]))
    fused = f0 * a_list[0] + f1 * a_list[1] + f2 * a_list[2]
    out_ref[...] = jnp.maximum(fused + f, 0.0)


def _final(feat, f0, agg, W_first, b_first, W_second, b_second, W_conv):
    RB = 2000
    grid = N_P // RB
    b_first2 = b_first.reshape(1, D)
    b_second2 = b_second.reshape(1, D)
    return pl.pallas_call(
        _final_body,
        grid=(grid,),
        in_specs=[
            pl.BlockSpec((RB, D), lambda i: (i, 0)),
            pl.BlockSpec((RB, D), lambda i: (i, 0)),
            pl.BlockSpec((NC, RB, DH), lambda i: (0, i, 0)),
            pl.BlockSpec((NC, RB, DH), lambda i: (0, i + N_P // RB, 0)),
            pl.BlockSpec((D, D), lambda i: (0, 0)),
            pl.BlockSpec((1, D), lambda i: (0, 0)),
            pl.BlockSpec((D, D), lambda i: (0, 0)),
            pl.BlockSpec((1, D), lambda i: (0, 0)),
            pl.BlockSpec((3, 3), lambda i: (0, 0)),
        ],
        out_specs=pl.BlockSpec((RB, D), lambda i: (i, 0)),
        out_shape=jax.ShapeDtypeStruct((N_P, D), jnp.float32),
    )(feat, f0, agg, agg, W_first, b_first2,
      W_second, b_second2, W_conv)


def kernel(feat, edge_weight, W_self, b_self, W_first, b_first, W_second,
           b_second, Wh1, bh1, Wh2, bh2, W_fusion, W_conv,
           protein_idx, hyperedge_idx, edge_type):
    pid = protein_idx.astype(jnp.int32)
    hid = hyperedge_idx.astype(jnp.int32)
    et = edge_type.astype(jnp.int32)
    w = edge_weight[:, 0]

    dst1 = hid + et * N_H
    s2 = pid + et * N_P
    dst1_2d = dst1.reshape(NS, NGR, NBUF, CH)
    s2_2d = s2.reshape(NS, NGR, NBUF, CH)
    featT = feat.reshape(NC * N_P, DH)

    pid2 = pid * 2
    gA = jnp.concatenate([pid2, pid2 + 1])
    gC = jnp.concatenate([dst1, dst1 + NROWSP])
    f0 = _feat0(feat, W_self, b_self)
    he2 = _seg_pass(featT, gA, dst1_2d, w)
    hew2 = _attn(he2, Wh1, bh1, Wh2.reshape(NHEADS, HD), bh2, W_fusion)
    agg = _seg_pass(hew2.reshape(NC * NROWSP, DH), gC, s2_2d, w)
    return _final(feat, f0, agg, W_first, b_first,
                  W_second, b_second, W_conv)
